# baseline (speedup 1.0000x reference)
import math

import jax
import jax.numpy as jnp
from jax.experimental import pallas as pl
from jax.experimental.pallas import tpu as pltpu

_HID = 256
_K = 100
_KP = 128
_N = 20000
_NP = 160 * 128
_HW = 64 * 64
_CH = 512
_CN = 2048
_HIGH = jax.lax.Precision.HIGHEST


def _tqg_kernel(scores_ref, boxes_ref, feat_ref, wp_ref, bp_ref,
                scores_out, centers_out, qc_out, qp_out, idx_scr):
    s = scores_ref[0]
    row = jax.lax.broadcasted_iota(jnp.int32, (160, 128), 0)
    col = jax.lax.broadcasted_iota(jnp.int32, (160, 128), 1)
    flat = row * 128 + col

    idx_scr[...] = jnp.zeros((_KP, 1), jnp.int32)

    def body(k, s):
        m = jnp.max(s)
        idx = jnp.min(jnp.where(s == m, flat, jnp.int32(2 ** 30)))
        scores_out[0, pl.ds(k, 1), :] = jnp.reshape(m, (1, 1))
        idx_scr[pl.ds(k, 1), :] = jnp.reshape(idx, (1, 1))
        return jnp.where(flat == idx, -jnp.inf, s)

    jax.lax.fori_loop(0, _K, body, s)

    idx_col = idx_scr[...]
    acc = jnp.zeros((4, _KP), jnp.float32)
    for c in range(_NP // _CN):
        iota = jax.lax.broadcasted_iota(jnp.int32, (_KP, _CN), 1) + c * _CN
        oh = (iota == idx_col).astype(jnp.float32)
        bc = boxes_ref[0, :, pl.ds(c * _CN, _CN)]
        acc = acc + jax.lax.dot_general(
            bc, oh, (((1,), (1,)), ((), ())),
            preferred_element_type=jnp.float32, precision=_HIGH)
    centers2 = (acc[0:2, :] + acc[2:4, :]) * 0.5

    eye = (jax.lax.broadcasted_iota(jnp.int32, (_KP, _KP), 0)
           == jax.lax.broadcasted_iota(jnp.int32, (_KP, _KP), 1)
           ).astype(jnp.float32)
    centers_col = jax.lax.dot_general(
        eye, centers2, (((1,), (1,)), ((), ())),
        preferred_element_type=jnp.float32, precision=_HIGH)
    centers_out[0] = centers_col

    cx = centers2[0:1, :]
    cy = centers2[1:2, :]
    gx = 2.0 * cx - 1.0
    gy = 2.0 * cy - 1.0
    x = jnp.clip((gx + 1.0) * 0.5 * 63.0, 0.0, 63.0)
    y = jnp.clip((gy + 1.0) * 0.5 * 63.0, 0.0, 63.0)
    x0f = jnp.floor(x)
    y0f = jnp.floor(y)
    wx = x - x0f
    wy = y - y0f
    x0 = x0f.astype(jnp.int32)
    y0 = y0f.astype(jnp.int32)
    x1 = jnp.clip(x0 + 1, 0, 63)
    y1 = jnp.clip(y0 + 1, 0, 63)
    x0 = jnp.clip(x0, 0, 63)
    y0 = jnp.clip(y0, 0, 63)
    p00 = y0 * 64 + x0
    p01 = y0 * 64 + x1
    p10 = y1 * 64 + x0
    p11 = y1 * 64 + x1
    w00 = (1.0 - wy) * (1.0 - wx)
    w01 = (1.0 - wy) * wx
    w10 = wy * (1.0 - wx)
    w11 = wy * wx

    qcT = jnp.zeros((_HID, _KP), jnp.float32)
    for h in range(0, _HW, _CH):
        hw = jax.lax.broadcasted_iota(jnp.int32, (_CH, _KP), 0) + h
        m = (jnp.where(hw == p00, w00, 0.0)
             + jnp.where(hw == p01, w01, 0.0)
             + jnp.where(hw == p10, w10, 0.0)
             + jnp.where(hw == p11, w11, 0.0))
        fc = feat_ref[0, :, pl.ds(h, _CH)]
        qcT = qcT + jax.lax.dot_general(
            fc, m, (((1,), (0,)), ((), ())),
            preferred_element_type=jnp.float32, precision=_HIGH)
    qc = jax.lax.dot_general(
        eye, qcT, (((1,), (1,)), ((), ())),
        preferred_element_type=jnp.float32, precision=_HIGH)
    qc_out[0] = qc

    cxc = centers_col[:, 0:1]
    cyc = centers_col[:, 1:2]
    L = jax.lax.broadcasted_iota(jnp.int32, (_KP, _HID), 1)
    l2 = jnp.bitwise_and(L, 127)
    tt = (2 * (l2 // 2)).astype(jnp.float32) / 128.0
    invd = jnp.exp(tt * (-math.log(10000.0)))
    coord = jnp.where(L < 128, cxc, cyc)
    pos = coord * (2.0 * math.pi) * invd
    pe = jnp.where(jnp.bitwise_and(l2, 1) == 0, jnp.sin(pos), jnp.cos(pos))
    qp = jax.lax.dot_general(
        pe, wp_ref[...], (((1,), (1,)), ((), ())),
        preferred_element_type=jnp.float32, precision=_HIGH)
    qp_out[0] = qp + bp_ref[...]


def kernel(features, teacher_boxes, teacher_scores, Wp, bp):
    B = features.shape[0]
    scores_p = jnp.pad(teacher_scores, ((0, 0), (0, _NP - _N)),
                       constant_values=-jnp.inf).reshape(B, 160, 128)
    boxes_t = jnp.pad(jnp.transpose(teacher_boxes, (0, 2, 1)),
                      ((0, 0), (0, 0), (0, _NP - _N)))
    feat = features.reshape(B, _HID, _HW)
    bp2 = bp.reshape(1, _HID)

    scores_o, centers_o, qc_o, qp_o = pl.pallas_call(
        _tqg_kernel,
        grid=(B,),
        in_specs=[
            pl.BlockSpec((1, 160, 128), lambda b: (b, 0, 0)),
            pl.BlockSpec((1, 4, _NP), lambda b: (b, 0, 0)),
            pl.BlockSpec((1, _HID, _HW), lambda b: (b, 0, 0)),
            pl.BlockSpec((_HID, _HID), lambda b: (0, 0)),
            pl.BlockSpec((1, _HID), lambda b: (0, 0)),
        ],
        out_specs=[
            pl.BlockSpec((1, _KP, 1), lambda b: (b, 0, 0)),
            pl.BlockSpec((1, _KP, 2), lambda b: (b, 0, 0)),
            pl.BlockSpec((1, _KP, _HID), lambda b: (b, 0, 0)),
            pl.BlockSpec((1, _KP, _HID), lambda b: (b, 0, 0)),
        ],
        out_shape=[
            jax.ShapeDtypeStruct((B, _KP, 1), jnp.float32),
            jax.ShapeDtypeStruct((B, _KP, 2), jnp.float32),
            jax.ShapeDtypeStruct((B, _KP, _HID), jnp.float32),
            jax.ShapeDtypeStruct((B, _KP, _HID), jnp.float32),
        ],
        scratch_shapes=[pltpu.VMEM((_KP, 1), jnp.int32)],
    )(scores_p, boxes_t, feat, Wp, bp2)

    topk_scores = scores_o[:, :_K, 0]
    box_centers = centers_o[:, :_K, :]
    query_content = qc_o[:, :_K, :]
    query_pos = qp_o[:, :_K, :]
    query_embed = jnp.concatenate([query_content, query_pos], axis=-1)
    return (query_embed, query_content, query_pos, box_centers, topk_scores)
